# baseline (speedup 1.0000x reference)
import functools
import math

import jax
import jax.numpy as jnp
from jax import lax
from jax.experimental import pallas as pl
from jax.experimental.pallas import tpu as pltpu
from jax.experimental.pallas import tpu_sc as plsc

_EMB = 32
_SCALE = math.sqrt(_EMB)

_NC = 2
_NS = 16
_NW = _NC * _NS

_B0 = 4096
_B1 = 200
_CHUNK = 1024
_NQ = (_B0 // 128) * (_B1 // 8)
_CPW = _NQ // _NW
_PITCH = 129


def _emb_kernel(tok_hbm, table_hbm, out_hbm, idx_v, rows_v, stage_v,
                gsem, wsem):
    wid = lax.axis_index("s") * _NC + lax.axis_index("c")
    q0 = wid * _CPW
    iota = lax.iota(jnp.int32, 16)
    jv0 = iota
    jv1 = iota + 16

    def start_gather(qi, buf):
        pltpu.sync_copy(tok_hbm.at[q0 + qi], idx_v.at[buf])

        @plsc.parallel_loop(0, _CHUNK // 16)
        def _prep(i):
            r = idx_v[buf, pl.ds(i * 16, 16)]
            hi = lax.shift_left(lax.shift_right_logical(r, 9), 9)
            mid = lax.shift_left(lax.bitwise_and(r, 127), 2)
            lo = lax.bitwise_and(lax.shift_right_logical(r, 7), 3)
            idx_v[buf, pl.ds(i * 16, 16)] = lax.bitwise_or(
                hi, lax.bitwise_or(mid, lo))

        pltpu.async_copy(table_hbm.at[idx_v.at[buf]], rows_v.at[buf],
                         gsem.at[buf])

    def wait_gather(buf):
        pltpu.make_async_copy(
            table_hbm.at[idx_v.at[buf]], rows_v.at[buf], gsem.at[buf]).wait()

    def drain_writes():
        for b in range(8):
            for jt in range(4):
                pltpu.make_async_copy(
                    stage_v.at[0, pl.ds(0, 8), pl.ds(0, 128)],
                    out_hbm.at[0, 0, 0],
                    wsem,
                ).wait()

    def do_chunk(qi, buf):
        q = q0 + qi
        a = lax.shift_right_logical(q, 5)
        c = lax.bitwise_and(q, 31)

        @pl.when(qi + 1 < _CPW)
        def _():
            start_gather(qi + 1, buf ^ 1)

        wait_gather(buf)

        @pl.when(qi > 0)
        def _():
            drain_writes()

        rows = rows_v.at[buf]

        @plsc.parallel_loop(0, 256)
        def _shuffle(i):
            for u in range(4):
                k = i * 4 + u
                b = lax.shift_right_logical(k, 7)
                d = lax.bitwise_and(k, 127)
                bv = jnp.full((16,), b, jnp.int32)
                dv = jnp.full((16,), d, jnp.int32)
                v0 = rows[k, pl.ds(0, 16)] * _SCALE
                v1 = rows[k, pl.ds(16, 16)] * _SCALE
                plsc.store_scatter(stage_v, [bv, jv0, dv], v0)
                plsc.store_scatter(stage_v, [bv, jv1, dv], v1)

        for b in range(8):
            for jt in range(4):
                pltpu.make_async_copy(
                    stage_v.at[b, pl.ds(jt * 8, 8), pl.ds(0, 128)],
                    out_hbm.at[a * 8 + b, jt, c],
                    wsem,
                ).start()

    start_gather(0, 0)

    def pair_body(g, carry):
        do_chunk(g * 2, 0)
        do_chunk(g * 2 + 1, 1)
        return carry

    lax.fori_loop(0, _CPW // 2, pair_body, 0)
    do_chunk(_CPW - 1, 0)
    drain_writes()


_TBLK = 512
_VOCAB = 1000000
_NTB = (_VOCAB + _TBLK - 1) // _TBLK
_VPAD = _NTB * 128


def _transpose_kernel(tt_ref, out_ref):
    x = tt_ref[...]
    parts = [x[:, lj * 128:(lj + 1) * 128].T for lj in range(4)]
    out_ref[...] = jnp.concatenate(parts, axis=1)


@jax.jit
def _table_packed(table_t):
    return pl.pallas_call(
        _transpose_kernel,
        grid=(_NTB,),
        in_specs=[pl.BlockSpec((_EMB, _TBLK), lambda g: (0, g))],
        out_specs=pl.BlockSpec((128, 128), lambda g: (g, 0)),
        out_shape=jax.ShapeDtypeStruct((_VPAD, 128), jnp.float32),
    )(table_t)


@jax.jit
def _lookup(tok_view, table):
    mesh = plsc.VectorSubcoreMesh(core_axis_name="c", subcore_axis_name="s")
    run = functools.partial(
        pl.kernel,
        mesh=mesh,
        out_type=jax.ShapeDtypeStruct((_B1, 4, 32, 8, 128), jnp.float32),
        scratch_types=[
            pltpu.VMEM((2, _CHUNK), jnp.int32),
            pltpu.VMEM((2, _CHUNK, _EMB), jnp.float32),
            pltpu.VMEM((8, 32, _PITCH), jnp.float32),
            pltpu.SemaphoreType.DMA((2,)),
            pltpu.SemaphoreType.DMA,
        ],
        compiler_params=pltpu.CompilerParams(
            use_tc_tiling_on_sc=False, needs_layout_passes=False),
    )(_emb_kernel)
    return run(tok_view, table)


def kernel(tokens, table):
    tok_view = (
        tokens.astype(jnp.int32)
        .T.reshape(25, 8, 32, 128)
        .transpose(0, 2, 1, 3)
        .reshape(_NQ, _CHUNK)
    )
    table_rm = _table_packed(table.T).reshape(_VPAD * 4, _EMB)
    out5 = _lookup(tok_view, table_rm)
    return (
        out5.transpose(2, 4, 0, 1, 3)
        .reshape(_B0, _B1, _EMB)
    )


# speedup vs baseline: 3.3878x; 3.3878x over previous
import functools
import math

import jax
import jax.numpy as jnp
from jax import lax
from jax.experimental import pallas as pl
from jax.experimental.pallas import tpu as pltpu
from jax.experimental.pallas import tpu_sc as plsc

_EMB = 32
_SCALE = math.sqrt(_EMB)

_NC = 2
_NS = 16
_NW = _NC * _NS

_B0 = 4096
_B1 = 200
_CHUNK = 1024
_NQ = (_B0 // 128) * (_B1 // 8)
_CPW = _NQ // _NW
_PITCH = 129


def _emb_kernel(tok_hbm, table_hbm, out_hbm, idx_v, rows_v, stage_v,
                gsem, wsem):
    wid = lax.axis_index("s") * _NC + lax.axis_index("c")
    q0 = wid * _CPW
    iota = lax.iota(jnp.int32, 16)
    jv0 = iota
    jv1 = iota + 16

    def start_gather(qi, buf):
        pltpu.sync_copy(tok_hbm.at[q0 + qi], idx_v.at[buf])

        @plsc.parallel_loop(0, _CHUNK // 16)
        def _prep(i):
            r = idx_v[buf, pl.ds(i * 16, 16)]
            hi = lax.shift_left(lax.shift_right_logical(r, 9), 9)
            mid = lax.shift_left(lax.bitwise_and(r, 127), 2)
            lo = lax.bitwise_and(lax.shift_right_logical(r, 7), 3)
            idx_v[buf, pl.ds(i * 16, 16)] = lax.bitwise_or(
                hi, lax.bitwise_or(mid, lo))

        pltpu.async_copy(table_hbm.at[idx_v.at[buf]], rows_v.at[buf],
                         gsem.at[buf])

    def wait_gather(buf):
        pltpu.make_async_copy(
            table_hbm.at[idx_v.at[buf]], rows_v.at[buf], gsem.at[buf]).wait()

    def drain_writes():
        for b in range(8):
            for jt in range(4):
                pltpu.make_async_copy(
                    stage_v.at[0, pl.ds(0, 8), pl.ds(0, 128)],
                    out_hbm.at[0, 0, 0],
                    wsem,
                ).wait()

    def do_chunk(qi, buf):
        q = q0 + qi
        a = lax.shift_right_logical(q, 5)
        c = lax.bitwise_and(q, 31)

        @pl.when(qi + 1 < _CPW)
        def _():
            start_gather(qi + 1, buf ^ 1)

        wait_gather(buf)

        @pl.when(qi > 0)
        def _():
            drain_writes()

        rows = rows_v.at[buf]

        @plsc.parallel_loop(0, 256)
        def _shuffle(i):
            for u in range(4):
                k = i * 4 + u
                b = lax.shift_right_logical(k, 7)
                d = lax.bitwise_and(k, 127)
                bv = jnp.full((16,), b, jnp.int32)
                dv = jnp.full((16,), d, jnp.int32)
                v0 = rows[k, pl.ds(0, 16)] * _SCALE
                v1 = rows[k, pl.ds(16, 16)] * _SCALE
                plsc.store_scatter(stage_v, [bv, jv0, dv], v0)
                plsc.store_scatter(stage_v, [bv, jv1, dv], v1)

        for b in range(8):
            for jt in range(4):
                pltpu.make_async_copy(
                    stage_v.at[b, pl.ds(jt * 8, 8), pl.ds(0, 128)],
                    out_hbm.at[a * 8 + b, jt, c],
                    wsem,
                ).start()

    start_gather(0, 0)

    def pair_body(g, carry):
        do_chunk(g * 2, 0)
        do_chunk(g * 2 + 1, 1)
        return carry

    lax.fori_loop(0, _CPW // 2, pair_body, 0)
    do_chunk(_CPW - 1, 0)
    drain_writes()


_TBLK = 8192
_VOCAB = 1000000
_NTB = (_VOCAB + _TBLK - 1) // _TBLK
_VPAD = _NTB * (_TBLK // 4)


def _transpose_kernel(tt_ref, out_ref):
    x = tt_ref[...]
    rows = []
    for s in range(_TBLK // 512):
        parts = [
            x[:, s * 512 + lj * 128: s * 512 + (lj + 1) * 128].T
            for lj in range(4)
        ]
        rows.append(jnp.concatenate(parts, axis=1))
    out_ref[...] = jnp.concatenate(rows, axis=0)


@jax.jit
def _table_packed(table_t):
    return pl.pallas_call(
        _transpose_kernel,
        grid=(_NTB,),
        in_specs=[pl.BlockSpec((_EMB, _TBLK), lambda g: (0, g))],
        out_specs=pl.BlockSpec((_TBLK // 4, 128), lambda g: (g, 0)),
        out_shape=jax.ShapeDtypeStruct((_VPAD, 128), jnp.float32),
    )(table_t)


@jax.jit
def _lookup(tok_view, table):
    mesh = plsc.VectorSubcoreMesh(core_axis_name="c", subcore_axis_name="s")
    run = functools.partial(
        pl.kernel,
        mesh=mesh,
        out_type=jax.ShapeDtypeStruct((_B1, 4, 32, 8, 128), jnp.float32),
        scratch_types=[
            pltpu.VMEM((2, _CHUNK), jnp.int32),
            pltpu.VMEM((2, _CHUNK, _EMB), jnp.float32),
            pltpu.VMEM((8, 32, _PITCH), jnp.float32),
            pltpu.SemaphoreType.DMA((2,)),
            pltpu.SemaphoreType.DMA,
        ],
        compiler_params=pltpu.CompilerParams(
            use_tc_tiling_on_sc=False, needs_layout_passes=False),
    )(_emb_kernel)
    return run(tok_view, table)


def kernel(tokens, table):
    tok_view = (
        tokens.astype(jnp.int32)
        .T.reshape(25, 8, 32, 128)
        .transpose(0, 2, 1, 3)
        .reshape(_NQ, _CHUNK)
    )
    table_rm = _table_packed(table.T).reshape(_VPAD * 4, _EMB)
    out5 = _lookup(tok_view, table_rm)
    return (
        out5.transpose(2, 4, 0, 1, 3)
        .reshape(_B0, _B1, _EMB)
    )
